# Initial kernel scaffold; baseline (speedup 1.0000x reference)
import jax
import jax.numpy as jnp
from jax.experimental import pallas as pl


def kernel(x, edge_index, W1, b1, W2, b2):
    raise NotImplementedError("write your pallas kernel here")


# speedup vs baseline: 24.5056x; 24.5056x over previous
import functools

import jax
import jax.numpy as jnp
from jax import lax
from jax.experimental import pallas as pl
from jax.experimental.pallas import tpu as pltpu
from jax.experimental.pallas import tpu_sc as plsc

_N = 10000
_NP = 10112
_NT = 16
_NC = 2
_NW = _NT * _NC
_RPT = _NP // _NT
_B = 128
_BN = 1000


def _make_sc_scatter(n_blocks: int, d: int):
  mesh = plsc.VectorSubcoreMesh(core_axis_name="c", subcore_axis_name="s")

  @functools.partial(
      pl.kernel,
      out_type=jax.ShapeDtypeStruct((_NC, _NP, d), jnp.float32),
      mesh=mesh,
      compiler_params=pltpu.CompilerParams(use_tc_tiling_on_sc=False),
      scratch_types=[
          pltpu.VMEM((n_blocks, _B), jnp.int32),
          pltpu.VMEM((n_blocks, _B), jnp.int32),
          pltpu.VMEM((_B, d), jnp.float32),
          pltpu.VMEM_SHARED((_NP, d), jnp.float32),
          pltpu.SemaphoreType.DMA,
      ],
  )
  def sc_scatter(table_hbm, src_hbm, dst_hbm, zeros_hbm, out_hbm,
                 src_v, dst_v, rows_v, acc, gsem):
    cid = lax.axis_index("c")
    sid = lax.axis_index("s")
    wid = sid * _NC + cid
    r0 = sid * _RPT
    pltpu.sync_copy(zeros_hbm.at[pl.ds(r0, _RPT)], acc.at[pl.ds(r0, _RPT)])
    pltpu.sync_copy(src_hbm.at[wid], src_v)
    pltpu.sync_copy(dst_hbm.at[wid], dst_v)
    plsc.subcore_barrier()

    def body(j, carry):
      pltpu.async_copy(table_hbm.at[src_v.at[j]], rows_v, gsem).wait()
      pltpu.sync_copy(rows_v, acc.at[dst_v.at[j]], add=True)
      return carry

    lax.fori_loop(0, n_blocks, body, 0)
    plsc.subcore_barrier()
    pltpu.sync_copy(acc.at[pl.ds(r0, _RPT)],
                    out_hbm.at[cid].at[pl.ds(r0, _RPT)])

  return sc_scatter


def _make_sc_degree(n_blocks: int):
  mesh = plsc.VectorSubcoreMesh(core_axis_name="c", subcore_axis_name="s")

  @functools.partial(
      pl.kernel,
      out_type=jax.ShapeDtypeStruct((_NC, _NP, 16), jnp.float32),
      mesh=mesh,
      compiler_params=pltpu.CompilerParams(use_tc_tiling_on_sc=False),
      scratch_types=[
          pltpu.VMEM((n_blocks, _B), jnp.int32),
          pltpu.VMEM((_B, 16), jnp.float32),
          pltpu.VMEM_SHARED((_NP, 16), jnp.float32),
      ],
  )
  def sc_degree(dst_hbm, ones_hbm, zeros_hbm, out_hbm, dst_v, ones_v, acc):
    cid = lax.axis_index("c")
    sid = lax.axis_index("s")
    wid = sid * _NC + cid
    r0 = sid * _RPT
    pltpu.sync_copy(zeros_hbm.at[pl.ds(r0, _RPT)], acc.at[pl.ds(r0, _RPT)])
    pltpu.sync_copy(dst_hbm.at[wid], dst_v)
    pltpu.sync_copy(ones_hbm, ones_v)
    plsc.subcore_barrier()

    def body(j, carry):
      pltpu.sync_copy(ones_v, acc.at[dst_v.at[j]], add=True)
      return carry

    lax.fori_loop(0, n_blocks, body, 0)
    plsc.subcore_barrier()
    pltpu.sync_copy(acc.at[pl.ds(r0, _RPT)],
                    out_hbm.at[cid].at[pl.ds(r0, _RPT)])

  return sc_degree


def _dinv_block(dp_ref):
  deg = 1.0 + dp_ref[0, :, 0:1] + dp_ref[1, :, 0:1]
  return lax.rsqrt(deg)


def _k2_body(x_ref, w1_ref, dp_ref, y1_ref):
  dinv = _dinv_block(dp_ref)
  xw = jnp.dot(x_ref[...], w1_ref[...], preferred_element_type=jnp.float32)
  y1_ref[...] = xw * dinv


def _k4_body(dp_ref, s1_ref, y1_ref, b1_ref, w2_ref, y2_ref):
  dinv = _dinv_block(dp_ref)
  t = dinv * (s1_ref[0] + s1_ref[1] + y1_ref[...]) + b1_ref[...]
  h = jnp.maximum(t, 0.0)
  y2_ref[...] = jnp.dot(h, w2_ref[...],
                        preferred_element_type=jnp.float32) * dinv


def _k6_body(dp_ref, s2_ref, y2_ref, b2_ref, o_ref):
  dinv = _dinv_block(dp_ref)
  o = dinv * (s2_ref[0] + s2_ref[1] + y2_ref[...]) + b2_ref[...]
  m = jnp.max(o, axis=1, keepdims=True)
  lse = m + jnp.log(jnp.sum(jnp.exp(o - m), axis=1, keepdims=True))
  o_ref[...] = o - lse


def kernel(x, edge_index, W1, b1, W2, b2):
  n, d_in = x.shape
  h = W1.shape[1]
  d_out = W2.shape[1]
  e = edge_index.shape[1]

  ept = -(-e // (_NW * _B)) * _B
  n_blocks = ept // _B
  pad = _NW * ept - e
  src3 = jnp.concatenate(
      [edge_index[0], jnp.full((pad,), _N, jnp.int32)]).reshape(_NW, n_blocks, _B)
  dst3 = jnp.concatenate(
      [edge_index[1], jnp.full((pad,), _N, jnp.int32)]).reshape(_NW, n_blocks, _B)

  ones16 = jnp.ones((_B, 16), jnp.float32)
  z16 = jnp.zeros((_NP, 16), jnp.float32)

  dp = _make_sc_degree(n_blocks)(dst3, ones16, z16)

  grid = (n // _BN,)
  y1 = pl.pallas_call(
      _k2_body,
      grid=grid,
      in_specs=[
          pl.BlockSpec((_BN, d_in), lambda i: (i, 0)),
          pl.BlockSpec((d_in, h), lambda i: (0, 0)),
          pl.BlockSpec((2, _BN, 16), lambda i: (0, i, 0)),
      ],
      out_specs=pl.BlockSpec((_BN, h), lambda i: (i, 0)),
      out_shape=jax.ShapeDtypeStruct((n, h), jnp.float32),
  )(x, W1, dp)

  y1p = jnp.concatenate([y1, jnp.zeros((_NP - n, h), jnp.float32)])
  s1 = _make_sc_scatter(n_blocks, h)(y1p, src3, dst3,
                                     jnp.zeros((_NP, h), jnp.float32))

  y2 = pl.pallas_call(
      _k4_body,
      grid=grid,
      in_specs=[
          pl.BlockSpec((2, _BN, 16), lambda i: (0, i, 0)),
          pl.BlockSpec((2, _BN, h), lambda i: (0, i, 0)),
          pl.BlockSpec((_BN, h), lambda i: (i, 0)),
          pl.BlockSpec((1, h), lambda i: (0, 0)),
          pl.BlockSpec((h, d_out), lambda i: (0, 0)),
      ],
      out_specs=pl.BlockSpec((_BN, d_out), lambda i: (i, 0)),
      out_shape=jax.ShapeDtypeStruct((n, d_out), jnp.float32),
  )(dp, s1, y1, b1.reshape(1, h), W2)

  y2p = jnp.concatenate([y2, jnp.zeros((_NP - n, d_out), jnp.float32)])
  s2 = _make_sc_scatter(n_blocks, d_out)(y2p, src3, dst3,
                                         jnp.zeros((_NP, d_out), jnp.float32))

  out = pl.pallas_call(
      _k6_body,
      grid=grid,
      in_specs=[
          pl.BlockSpec((2, _BN, 16), lambda i: (0, i, 0)),
          pl.BlockSpec((2, _BN, d_out), lambda i: (0, i, 0)),
          pl.BlockSpec((_BN, d_out), lambda i: (i, 0)),
          pl.BlockSpec((1, d_out), lambda i: (0, 0)),
      ],
      out_specs=pl.BlockSpec((_BN, d_out), lambda i: (i, 0)),
      out_shape=jax.ShapeDtypeStruct((n, d_out), jnp.float32),
  )(dp, s2, y2, b2.reshape(1, d_out))

  return out
